# Initial kernel scaffold; baseline (speedup 1.0000x reference)
import jax
import jax.numpy as jnp
from jax.experimental import pallas as pl


def kernel(x, edge_index, batch_vec, W1, b1, W2, b2, lw1, lb1, lw2, lb2):
    raise NotImplementedError("write your pallas kernel here")


# speedup vs baseline: 31.6152x; 31.6152x over previous
import functools

import jax
import jax.numpy as jnp
from jax import lax
from jax.experimental import pallas as pl
from jax.experimental.pallas import tpu as pltpu
from jax.experimental.pallas import tpu_sc as plsc

_N = 10000
_E = 320000
_D = 128
_H = 16
_HID = 100
_NG = 64
_NC = 10

_CH = 128
_NCHUNK = _E // _CH
_NW = 32
_MAXC = -(-_NCHUNK // _NW)
_BASEC = _NCHUNK // _NW
_EXTRA = _NCHUNK - _BASEC * _NW
_NCHUNK_PAD = 2504
_WIN = 88
_RPS = 632

_mesh = plsc.VectorSubcoreMesh(core_axis_name="c", subcore_axis_name="s")
_sc_params = pltpu.CompilerParams(use_tc_tiling_on_sc=False)


def _tile_ranges(w):
    cnt = jnp.where(w < _EXTRA, _MAXC, _BASEC)
    start = _BASEC * w + jnp.minimum(w, _EXTRA)
    dma_start = (start // 8) * 8
    loff = start - dma_start
    return cnt, dma_start, loff


def _zero_fill(ref, nrows):
    z = jnp.zeros((16,), jnp.float32)

    def body(i, _):
        ref[i] = z
        return 0

    lax.fori_loop(0, nrows, body, 0)


def _slab_start(s):
    return jnp.minimum(s * _RPS, _N - _RPS)


def _scatter_epilogue(acc_sh, out_hbm, c, s):
    plsc.subcore_barrier()
    r0 = _slab_start(s)
    pltpu.sync_copy(acc_sh.at[pl.ds(r0, _RPS)], out_hbm.at[c, pl.ds(r0, _RPS)])


@functools.partial(
    pl.kernel,
    out_type=jax.ShapeDtypeStruct((2, _N, _H), jnp.float32),
    mesh=_mesh,
    scratch_types=[
        pltpu.VMEM((_WIN, _CH), jnp.int32),
        pltpu.VMEM((_CH, _H), jnp.float32),
        pltpu.VMEM((_RPS, _H), jnp.float32),
        pltpu.VMEM_SHARED((_N, _H), jnp.float32),
        pltpu.SemaphoreType.DMA,
    ],
    compiler_params=_sc_params,
)
def _sc_degree(dst_hbm, out_hbm, dst_v, ones_v, zrows_v, acc_sh, sem):
    c = lax.axis_index("c")
    s = lax.axis_index("s")
    w = s * 2 + c
    cnt, dma_start, loff = _tile_ranges(w)

    _zero_fill(zrows_v, _RPS)
    one = jnp.ones((16,), jnp.float32)

    def fill_ones(i, _):
        ones_v[i] = one
        return 0

    lax.fori_loop(0, _CH, fill_ones, 0)
    pltpu.sync_copy(zrows_v, acc_sh.at[pl.ds(_slab_start(s), _RPS)])
    pltpu.async_copy(dst_hbm.at[pl.ds(dma_start, _WIN)], dst_v, sem).wait()
    plsc.subcore_barrier()

    def step(k, _):
        pltpu.sync_copy(ones_v, acc_sh.at[dst_v.at[loff + k]], add=True)
        return 0

    lax.fori_loop(0, cnt, step, 0)
    _scatter_epilogue(acc_sh, out_hbm, c, s)


@functools.partial(
    pl.kernel,
    out_type=jax.ShapeDtypeStruct((2, _N, _H), jnp.float32),
    mesh=_mesh,
    scratch_types=[
        pltpu.VMEM((_WIN, _CH), jnp.int32),
        pltpu.VMEM((_WIN, _CH), jnp.int32),
        pltpu.VMEM((_CH, _H), jnp.float32),
        pltpu.VMEM((_RPS, _H), jnp.float32),
        pltpu.VMEM_SHARED((_N, _H), jnp.float32),
        pltpu.SemaphoreType.DMA,
        pltpu.SemaphoreType.DMA,
    ],
    compiler_params=_sc_params,
)
def _sc_aggregate(hs_hbm, src_hbm, dst_hbm, out_hbm, src_v, dst_v, rows_v,
                  zrows_v, acc_sh, gsem, isem):
    c = lax.axis_index("c")
    s = lax.axis_index("s")
    w = s * 2 + c
    cnt, dma_start, loff = _tile_ranges(w)

    _zero_fill(zrows_v, _RPS)
    pltpu.sync_copy(zrows_v, acc_sh.at[pl.ds(_slab_start(s), _RPS)])
    pltpu.async_copy(src_hbm.at[pl.ds(dma_start, _WIN)], src_v, isem)
    pltpu.async_copy(dst_hbm.at[pl.ds(dma_start, _WIN)], dst_v, isem)
    pltpu.make_async_copy(src_hbm.at[pl.ds(dma_start, _WIN)], src_v, isem).wait()
    pltpu.make_async_copy(dst_hbm.at[pl.ds(dma_start, _WIN)], dst_v, isem).wait()
    plsc.subcore_barrier()

    def step(k, _):
        row = loff + k
        pltpu.async_copy(hs_hbm.at[src_v.at[row]], rows_v, gsem).wait()
        pltpu.sync_copy(rows_v, acc_sh.at[dst_v.at[row]], add=True)
        return 0

    lax.fori_loop(0, cnt, step, 0)
    _scatter_epilogue(acc_sh, out_hbm, c, s)


_RB = 1000
_NGRID = _N // _RB

_hp = jax.lax.Precision.HIGHEST


def _tc1_body(x_ref, w1_ref, d0_ref, d1_ref, hs_ref, dinv_ref):
    deg = 1.0 + d0_ref[...] + d1_ref[...]
    dinv = lax.rsqrt(deg)
    h = jnp.dot(x_ref[...], w1_ref[...], preferred_element_type=jnp.float32,
                precision=_hp)
    hs_ref[...] = h * dinv
    dinv_ref[...] = dinv


def _tc2_body(a0_ref, a1_ref, hs_ref, dinv_ref, b1_ref, w2_ref, hs2_ref):
    dinv = dinv_ref[...]
    t = dinv * (a0_ref[...] + a1_ref[...] + hs_ref[...]) + b1_ref[...]
    t = jnp.maximum(t, 0.0)
    hs2_ref[...] = jnp.dot(t, w2_ref[...], preferred_element_type=jnp.float32,
                           precision=_hp) * dinv


def _tc3_body(a0_ref, a1_ref, hs_ref, dinv_ref, b2_ref, bv_ref, lw1_ref,
              lb1_ref, lw2_ref, lb2_ref, out_ref, acc_ref):
    i = pl.program_id(0)
    t = dinv_ref[...] * (a0_ref[...] + a1_ref[...] + hs_ref[...]) + b2_ref[...]
    t = jnp.maximum(t, 0.0)
    gids = lax.broadcasted_iota(jnp.int32, (_RB, _NG), 1)
    oh = (bv_ref[...] == gids).astype(jnp.float32)
    contrib = lax.dot_general(oh, t, (((0,), (0,)), ((), ())),
                              preferred_element_type=jnp.float32,
                              precision=_hp)

    @pl.when(i == 0)
    def _():
        acc_ref[...] = contrib

    @pl.when(i > 0)
    def _():
        acc_ref[...] = acc_ref[...] + contrib

    @pl.when(i == _NGRID - 1)
    def _():
        r = jnp.maximum(acc_ref[...], 0.0)
        q = jnp.maximum(
            jnp.dot(r, lw1_ref[...], preferred_element_type=jnp.float32,
                    precision=_hp) + lb1_ref[...], 0.0)
        out_ref[...] = jnp.dot(q, lw2_ref[...],
                               preferred_element_type=jnp.float32,
                               precision=_hp) + lb2_ref[...]


def _row_spec(cols):
    return pl.BlockSpec((_RB, cols), lambda i: (i, 0))


def _full_spec(r, cols):
    return pl.BlockSpec((r, cols), lambda i: (0, 0))


_tc1 = pl.pallas_call(
    _tc1_body,
    grid=(_NGRID,),
    in_specs=[_row_spec(_D), _full_spec(_D, _H), _row_spec(_H), _row_spec(_H)],
    out_specs=[_row_spec(_H), _row_spec(_H)],
    out_shape=[jax.ShapeDtypeStruct((_N, _H), jnp.float32),
               jax.ShapeDtypeStruct((_N, _H), jnp.float32)],
)

_tc2 = pl.pallas_call(
    _tc2_body,
    grid=(_NGRID,),
    in_specs=[_row_spec(_H), _row_spec(_H), _row_spec(_H), _row_spec(_H),
              _full_spec(1, _H), _full_spec(_H, _H)],
    out_specs=_row_spec(_H),
    out_shape=jax.ShapeDtypeStruct((_N, _H), jnp.float32),
)

_tc3 = pl.pallas_call(
    _tc3_body,
    grid=(_NGRID,),
    in_specs=[_row_spec(_H), _row_spec(_H), _row_spec(_H), _row_spec(_H),
              _full_spec(1, _H), _row_spec(1), _full_spec(_H, _HID),
              _full_spec(1, _HID), _full_spec(_HID, _NC), _full_spec(1, _NC)],
    out_specs=_full_spec(_NG, _NC),
    out_shape=jax.ShapeDtypeStruct((_NG, _NC), jnp.float32),
    scratch_shapes=[pltpu.VMEM((_NG, _H), jnp.float32)],
)


def kernel(x, edge_index, batch_vec, W1, b1, W2, b2, lw1, lb1, lw2, lb2):
    pad = ((0, _NCHUNK_PAD - _NCHUNK), (0, 0))
    src2d = jnp.pad(edge_index[0].astype(jnp.int32).reshape(_NCHUNK, _CH), pad)
    dst2d = jnp.pad(edge_index[1].astype(jnp.int32).reshape(_NCHUNK, _CH), pad)

    degp = _sc_degree(dst2d)
    hs1, dinv = _tc1(x, W1, degp[0], degp[1])
    a1 = _sc_aggregate(hs1, src2d, dst2d)
    hs2 = _tc2(a1[0], a1[1], hs1, dinv, b1.reshape(1, _H), W2)
    a2 = _sc_aggregate(hs2, src2d, dst2d)
    out = _tc3(a2[0], a2[1], hs2, dinv, b2.reshape(1, _H),
               batch_vec.astype(jnp.int32).reshape(_N, 1), lw1,
               lb1.reshape(1, _HID), lw2, lb2.reshape(1, _NC))
    return out


# speedup vs baseline: 36.7498x; 1.1624x over previous
import functools

import jax
import jax.numpy as jnp
from jax import lax
from jax.experimental import pallas as pl
from jax.experimental.pallas import tpu as pltpu
from jax.experimental.pallas import tpu_sc as plsc

_N = 10000
_E = 320000
_D = 128
_H = 16
_HID = 100
_NG = 64
_NC = 10

_CH = 128
_NCHUNK = _E // _CH
_NW = 32
_MAXC = -(-_NCHUNK // _NW)
_BASEC = _NCHUNK // _NW
_EXTRA = _NCHUNK - _BASEC * _NW
_NCHUNK_PAD = 2504
_WIN = 88
_RPS = 632
_NBUF = 4

_mesh = plsc.VectorSubcoreMesh(core_axis_name="c", subcore_axis_name="s")
_sc_params = pltpu.CompilerParams(use_tc_tiling_on_sc=False)


def _tile_ranges(w):
    cnt = jnp.where(w < _EXTRA, _MAXC, _BASEC)
    start = _BASEC * w + jnp.minimum(w, _EXTRA)
    dma_start = (start // 8) * 8
    loff = start - dma_start
    return cnt, dma_start, loff


def _zero_fill(ref, nrows):
    z = jnp.zeros((16,), jnp.float32)

    def body(i, _):
        ref[i] = z
        return 0

    lax.fori_loop(0, nrows, body, 0)


def _slab_start(s):
    return jnp.minimum(s * _RPS, _N - _RPS)


def _scatter_epilogue(acc_sh, out_hbm, c, s):
    plsc.subcore_barrier()
    r0 = _slab_start(s)
    pltpu.sync_copy(acc_sh.at[pl.ds(r0, _RPS)], out_hbm.at[c, pl.ds(r0, _RPS)])


@functools.partial(
    pl.kernel,
    out_type=jax.ShapeDtypeStruct((2, _N, _H), jnp.float32),
    mesh=_mesh,
    scratch_types=[
        pltpu.VMEM((_WIN, _CH), jnp.int32),
        pltpu.VMEM((_CH, _H), jnp.float32),
        pltpu.VMEM((_RPS, _H), jnp.float32),
        pltpu.VMEM_SHARED((_N, _H), jnp.float32),
        pltpu.SemaphoreType.DMA,
    ],
    compiler_params=_sc_params,
)
def _sc_degree(dst_hbm, out_hbm, dst_v, ones_v, zrows_v, acc_sh, sem):
    c = lax.axis_index("c")
    s = lax.axis_index("s")
    w = s * 2 + c
    cnt, dma_start, loff = _tile_ranges(w)

    _zero_fill(zrows_v, _RPS)
    one = jnp.ones((16,), jnp.float32)

    def fill_ones(i, _):
        ones_v[i] = one
        return 0

    lax.fori_loop(0, _CH, fill_ones, 0)
    pltpu.sync_copy(zrows_v, acc_sh.at[pl.ds(_slab_start(s), _RPS)])
    pltpu.async_copy(dst_hbm.at[pl.ds(dma_start, _WIN)], dst_v, sem).wait()
    plsc.subcore_barrier()

    def step(k, _):
        pltpu.async_copy(ones_v, acc_sh.at[dst_v.at[loff + k]], sem, add=True)
        return 0

    lax.fori_loop(0, cnt, step, 0)

    def drain(k, _):
        pltpu.make_async_copy(ones_v, acc_sh.at[dst_v.at[loff + k]], sem).wait()
        return 0

    lax.fori_loop(0, cnt, drain, 0)
    _scatter_epilogue(acc_sh, out_hbm, c, s)


@functools.partial(
    pl.kernel,
    out_type=jax.ShapeDtypeStruct((2, _N, _H), jnp.float32),
    mesh=_mesh,
    scratch_types=[
        pltpu.VMEM((_WIN, _CH), jnp.int32),
        pltpu.VMEM((_WIN, _CH), jnp.int32),
        pltpu.VMEM((_NBUF, _CH, _H), jnp.float32),
        pltpu.VMEM((_RPS, _H), jnp.float32),
        pltpu.VMEM_SHARED((_N, _H), jnp.float32),
        pltpu.SemaphoreType.DMA,
        pltpu.SemaphoreType.DMA,
        pltpu.SemaphoreType.DMA,
    ],
    compiler_params=_sc_params,
)
def _sc_aggregate(hs_hbm, src_hbm, dst_hbm, out_hbm, src_v, dst_v, rows_v,
                  zrows_v, acc_sh, gsem, ssem, isem):
    c = lax.axis_index("c")
    s = lax.axis_index("s")
    w = s * 2 + c
    cnt, dma_start, loff = _tile_ranges(w)

    _zero_fill(zrows_v, _RPS)
    pltpu.sync_copy(zrows_v, acc_sh.at[pl.ds(_slab_start(s), _RPS)])
    pltpu.async_copy(src_hbm.at[pl.ds(dma_start, _WIN)], src_v, isem)
    pltpu.async_copy(dst_hbm.at[pl.ds(dma_start, _WIN)], dst_v, isem)
    pltpu.make_async_copy(src_hbm.at[pl.ds(dma_start, _WIN)], src_v, isem).wait()
    pltpu.make_async_copy(dst_hbm.at[pl.ds(dma_start, _WIN)], dst_v, isem).wait()
    plsc.subcore_barrier()

    def gissue(j):
        pltpu.async_copy(hs_hbm.at[src_v.at[loff + j]],
                         rows_v.at[j % _NBUF], gsem)

    def gwait(j):
        pltpu.make_async_copy(hs_hbm.at[src_v.at[loff + j]],
                              rows_v.at[j % _NBUF], gsem).wait()

    def sissue(j):
        pltpu.async_copy(rows_v.at[j % _NBUF],
                         acc_sh.at[dst_v.at[loff + j]], ssem, add=True)

    def swait(j):
        pltpu.make_async_copy(rows_v.at[j % _NBUF],
                              acc_sh.at[dst_v.at[loff + j]], ssem).wait()

    gissue(0)
    gissue(1)

    def step(k, _):
        gwait(k)
        sissue(k)

        @pl.when(k < 2)
        def _():
            gissue(k + 2)

        @pl.when(jnp.logical_and(k >= 2, k + 2 < cnt))
        def _():
            swait(k - 2)
            gissue(k + 2)

        return 0

    lax.fori_loop(0, cnt, step, 0)

    def drain(j, _):
        swait(j)
        return 0

    lax.fori_loop(cnt - 4, cnt, drain, 0)
    _scatter_epilogue(acc_sh, out_hbm, c, s)


_RB = 1000
_NGRID = _N // _RB

_hp = jax.lax.Precision.HIGHEST


def _tc0_body(x_ref, w1_ref, h_ref):
    h_ref[...] = jnp.dot(x_ref[...], w1_ref[...],
                         preferred_element_type=jnp.float32, precision=_hp)


def _tc1_body(h_ref, d0_ref, d1_ref, hs_ref, dinv_ref):
    deg = 1.0 + d0_ref[...] + d1_ref[...]
    dinv = lax.rsqrt(deg)
    hs_ref[...] = h_ref[...] * dinv
    dinv_ref[...] = dinv


def _tc2_body(a0_ref, a1_ref, hs_ref, dinv_ref, b1_ref, w2_ref, hs2_ref):
    dinv = dinv_ref[...]
    t = dinv * (a0_ref[...] + a1_ref[...] + hs_ref[...]) + b1_ref[...]
    t = jnp.maximum(t, 0.0)
    hs2_ref[...] = jnp.dot(t, w2_ref[...], preferred_element_type=jnp.float32,
                           precision=_hp) * dinv


def _tc3_body(a0_ref, a1_ref, hs_ref, dinv_ref, b2_ref, bv_ref, lw1_ref,
              lb1_ref, lw2_ref, lb2_ref, out_ref, acc_ref):
    i = pl.program_id(0)
    t = dinv_ref[...] * (a0_ref[...] + a1_ref[...] + hs_ref[...]) + b2_ref[...]
    t = jnp.maximum(t, 0.0)
    gids = lax.broadcasted_iota(jnp.int32, (_RB, _NG), 1)
    oh = (bv_ref[...] == gids).astype(jnp.float32)
    contrib = lax.dot_general(oh, t, (((0,), (0,)), ((), ())),
                              preferred_element_type=jnp.float32,
                              precision=_hp)

    @pl.when(i == 0)
    def _():
        acc_ref[...] = contrib

    @pl.when(i > 0)
    def _():
        acc_ref[...] = acc_ref[...] + contrib

    @pl.when(i == _NGRID - 1)
    def _():
        r = jnp.maximum(acc_ref[...], 0.0)
        q = jnp.maximum(
            jnp.dot(r, lw1_ref[...], preferred_element_type=jnp.float32,
                    precision=_hp) + lb1_ref[...], 0.0)
        out_ref[...] = jnp.dot(q, lw2_ref[...],
                               preferred_element_type=jnp.float32,
                               precision=_hp) + lb2_ref[...]


def _row_spec(cols):
    return pl.BlockSpec((_RB, cols), lambda i: (i, 0))


def _full_spec(r, cols):
    return pl.BlockSpec((r, cols), lambda i: (0, 0))


_tc0 = pl.pallas_call(
    _tc0_body,
    grid=(_NGRID,),
    in_specs=[_row_spec(_D), _full_spec(_D, _H)],
    out_specs=_row_spec(_H),
    out_shape=jax.ShapeDtypeStruct((_N, _H), jnp.float32),
)

_tc1 = pl.pallas_call(
    _tc1_body,
    grid=(_NGRID,),
    in_specs=[_row_spec(_H), _row_spec(_H), _row_spec(_H)],
    out_specs=[_row_spec(_H), _row_spec(_H)],
    out_shape=[jax.ShapeDtypeStruct((_N, _H), jnp.float32),
               jax.ShapeDtypeStruct((_N, _H), jnp.float32)],
)

_tc2 = pl.pallas_call(
    _tc2_body,
    grid=(_NGRID,),
    in_specs=[_row_spec(_H), _row_spec(_H), _row_spec(_H), _row_spec(_H),
              _full_spec(1, _H), _full_spec(_H, _H)],
    out_specs=_row_spec(_H),
    out_shape=jax.ShapeDtypeStruct((_N, _H), jnp.float32),
)

_tc3 = pl.pallas_call(
    _tc3_body,
    grid=(_NGRID,),
    in_specs=[_row_spec(_H), _row_spec(_H), _row_spec(_H), _row_spec(_H),
              _full_spec(1, _H), _row_spec(1), _full_spec(_H, _HID),
              _full_spec(1, _HID), _full_spec(_HID, _NC), _full_spec(1, _NC)],
    out_specs=_full_spec(_NG, _NC),
    out_shape=jax.ShapeDtypeStruct((_NG, _NC), jnp.float32),
    scratch_shapes=[pltpu.VMEM((_NG, _H), jnp.float32)],
)


def kernel(x, edge_index, batch_vec, W1, b1, W2, b2, lw1, lb1, lw2, lb2):
    pad = ((0, _NCHUNK_PAD - _NCHUNK), (0, 0))
    src2d = jnp.pad(edge_index[0].astype(jnp.int32).reshape(_NCHUNK, _CH), pad)
    dst2d = jnp.pad(edge_index[1].astype(jnp.int32).reshape(_NCHUNK, _CH), pad)

    degp = _sc_degree(dst2d)
    h1 = _tc0(x, W1)
    hs1, dinv = _tc1(h1, degp[0], degp[1])
    a1 = _sc_aggregate(hs1, src2d, dst2d)
    hs2 = _tc2(a1[0], a1[1], hs1, dinv, b1.reshape(1, _H), W2)
    a2 = _sc_aggregate(hs2, src2d, dst2d)
    out = _tc3(a2[0], a2[1], hs2, dinv, b2.reshape(1, _H),
               batch_vec.astype(jnp.int32).reshape(_N, 1), lw1,
               lb1.reshape(1, _HID), lw2, lb2.reshape(1, _NC))
    return out


# speedup vs baseline: 40.0695x; 1.0903x over previous
import functools

import jax
import jax.numpy as jnp
from jax import lax
from jax.experimental import pallas as pl
from jax.experimental.pallas import tpu as pltpu
from jax.experimental.pallas import tpu_sc as plsc

_N = 10000
_E = 320000
_D = 128
_H = 16
_HID = 100
_NG = 64
_NC = 10

_CH = 128
_NCHUNK = _E // _CH
_NW = 32
_MAXC = -(-_NCHUNK // _NW)
_BASEC = _NCHUNK // _NW
_EXTRA = _NCHUNK - _BASEC * _NW
_WIN = _MAXC
_RPS = 632
_NBUF = 4

_mesh = plsc.VectorSubcoreMesh(core_axis_name="c", subcore_axis_name="s")
_sc_params = pltpu.CompilerParams(use_tc_tiling_on_sc=False)


def _tile_ranges(w):
    cnt = jnp.where(w < _EXTRA, _MAXC, _BASEC)
    start = _BASEC * w + jnp.minimum(w, _EXTRA)
    dma_start = jnp.minimum(start, _NCHUNK - _WIN)
    loff = start - dma_start
    return cnt, dma_start, loff


def _zero_fill(ref, nrows):
    z = jnp.zeros((16,), jnp.float32)

    def body(i, _):
        ref[i] = z
        return 0

    lax.fori_loop(0, nrows, body, 0)


def _slab_start(s):
    return jnp.minimum(s * _RPS, _N - _RPS)


def _scatter_epilogue(acc_sh, out_hbm, c, s):
    plsc.subcore_barrier()
    r0 = _slab_start(s)
    pltpu.sync_copy(acc_sh.at[pl.ds(r0, _RPS)], out_hbm.at[c, pl.ds(r0, _RPS)])


@functools.partial(
    pl.kernel,
    out_type=jax.ShapeDtypeStruct((2, _N, _H), jnp.float32),
    mesh=_mesh,
    scratch_types=[
        pltpu.VMEM((_WIN, _CH), jnp.int32),
        pltpu.VMEM((_CH, _H), jnp.float32),
        pltpu.VMEM((_RPS, _H), jnp.float32),
        pltpu.VMEM_SHARED((_N, _H), jnp.float32),
        pltpu.SemaphoreType.DMA,
    ],
    compiler_params=_sc_params,
)
def _sc_degree(dst_hbm, out_hbm, dst_v, ones_v, zrows_v, acc_sh, sem):
    c = lax.axis_index("c")
    s = lax.axis_index("s")
    w = s * 2 + c
    cnt, dma_start, loff = _tile_ranges(w)

    _zero_fill(zrows_v, _RPS)
    one = jnp.ones((16,), jnp.float32)

    def fill_ones(i, _):
        ones_v[i] = one
        return 0

    lax.fori_loop(0, _CH, fill_ones, 0)
    pltpu.sync_copy(zrows_v, acc_sh.at[pl.ds(_slab_start(s), _RPS)])
    pltpu.async_copy(dst_hbm.at[pl.ds(dma_start, _WIN)], dst_v, sem).wait()
    plsc.subcore_barrier()

    def step(k, _):
        pltpu.async_copy(ones_v, acc_sh.at[dst_v.at[loff + k]], sem, add=True)
        return 0

    lax.fori_loop(0, cnt, step, 0)

    def drain(k, _):
        pltpu.make_async_copy(ones_v, acc_sh.at[dst_v.at[loff + k]], sem).wait()
        return 0

    lax.fori_loop(0, cnt, drain, 0)
    _scatter_epilogue(acc_sh, out_hbm, c, s)


@functools.partial(
    pl.kernel,
    out_type=jax.ShapeDtypeStruct((2, _N, _H), jnp.float32),
    mesh=_mesh,
    scratch_types=[
        pltpu.VMEM((_WIN, _CH), jnp.int32),
        pltpu.VMEM((_WIN, _CH), jnp.int32),
        pltpu.VMEM((_NBUF, _CH, _H), jnp.float32),
        pltpu.VMEM((_RPS, _H), jnp.float32),
        pltpu.VMEM_SHARED((_N, _H), jnp.float32),
        pltpu.SemaphoreType.DMA,
        pltpu.SemaphoreType.DMA,
        pltpu.SemaphoreType.DMA,
    ],
    compiler_params=_sc_params,
)
def _sc_aggregate(hs_hbm, src_hbm, dst_hbm, out_hbm, src_v, dst_v, rows_v,
                  zrows_v, acc_sh, gsem, ssem, isem):
    c = lax.axis_index("c")
    s = lax.axis_index("s")
    w = s * 2 + c
    cnt, dma_start, loff = _tile_ranges(w)

    _zero_fill(zrows_v, _RPS)
    pltpu.sync_copy(zrows_v, acc_sh.at[pl.ds(_slab_start(s), _RPS)])
    pltpu.async_copy(src_hbm.at[pl.ds(dma_start, _WIN)], src_v, isem)
    pltpu.async_copy(dst_hbm.at[pl.ds(dma_start, _WIN)], dst_v, isem)
    pltpu.make_async_copy(src_hbm.at[pl.ds(dma_start, _WIN)], src_v, isem).wait()
    pltpu.make_async_copy(dst_hbm.at[pl.ds(dma_start, _WIN)], dst_v, isem).wait()
    plsc.subcore_barrier()

    def gissue(j):
        pltpu.async_copy(hs_hbm.at[src_v.at[loff + j]],
                         rows_v.at[j % _NBUF], gsem)

    def gwait(j):
        pltpu.make_async_copy(hs_hbm.at[src_v.at[loff + j]],
                              rows_v.at[j % _NBUF], gsem).wait()

    def sissue(j):
        pltpu.async_copy(rows_v.at[j % _NBUF],
                         acc_sh.at[dst_v.at[loff + j]], ssem, add=True)

    def swait(j):
        pltpu.make_async_copy(rows_v.at[j % _NBUF],
                              acc_sh.at[dst_v.at[loff + j]], ssem).wait()

    gissue(0)
    gissue(1)

    def step(k, _):
        gwait(k)
        sissue(k)

        @pl.when(k < 2)
        def _():
            gissue(k + 2)

        @pl.when(jnp.logical_and(k >= 2, k + 2 < cnt))
        def _():
            swait(k - 2)
            gissue(k + 2)

        return 0

    lax.fori_loop(0, cnt, step, 0)

    def drain(j, _):
        swait(j)
        return 0

    lax.fori_loop(cnt - 4, cnt, drain, 0)
    _scatter_epilogue(acc_sh, out_hbm, c, s)


_RB = 1000
_NGRID = _N // _RB

_hp = jax.lax.Precision.HIGHEST


def _tc0_body(x_ref, w1_ref, h_ref):
    h_ref[...] = jnp.dot(x_ref[...], w1_ref[...],
                         preferred_element_type=jnp.float32, precision=_hp)


def _tc1_body(h_ref, dp_ref, hs_ref, dinv_ref):
    deg = 1.0 + dp_ref[0] + dp_ref[1]
    dinv = lax.rsqrt(deg)
    hs_ref[...] = h_ref[...] * dinv
    dinv_ref[...] = dinv


def _tc2_body(ap_ref, hs_ref, dinv_ref, b1_ref, w2_ref, hs2_ref):
    dinv = dinv_ref[...]
    t = dinv * (ap_ref[0] + ap_ref[1] + hs_ref[...]) + b1_ref[...]
    t = jnp.maximum(t, 0.0)
    hs2_ref[...] = jnp.dot(t, w2_ref[...], preferred_element_type=jnp.float32,
                           precision=_hp) * dinv


def _tc3_body(ap_ref, hs_ref, dinv_ref, b2_ref, bv_ref, lw1_ref,
              lb1_ref, lw2_ref, lb2_ref, out_ref, acc_ref):
    i = pl.program_id(0)
    t = dinv_ref[...] * (ap_ref[0] + ap_ref[1] + hs_ref[...]) + b2_ref[...]
    t = jnp.maximum(t, 0.0)
    gids = lax.broadcasted_iota(jnp.int32, (_RB, _NG), 1)
    oh = (bv_ref[...] == gids).astype(jnp.float32)
    contrib = lax.dot_general(oh, t, (((0,), (0,)), ((), ())),
                              preferred_element_type=jnp.float32,
                              precision=_hp)

    @pl.when(i == 0)
    def _():
        acc_ref[...] = contrib

    @pl.when(i > 0)
    def _():
        acc_ref[...] = acc_ref[...] + contrib

    @pl.when(i == _NGRID - 1)
    def _():
        r = jnp.maximum(acc_ref[...], 0.0)
        q = jnp.maximum(
            jnp.dot(r, lw1_ref[...], preferred_element_type=jnp.float32,
                    precision=_hp) + lb1_ref[...], 0.0)
        out_ref[...] = jnp.dot(q, lw2_ref[...],
                               preferred_element_type=jnp.float32,
                               precision=_hp) + lb2_ref[...]


def _row_spec(cols):
    return pl.BlockSpec((_RB, cols), lambda i: (i, 0))


_pair_spec = pl.BlockSpec((2, _RB, _H), lambda i: (0, i, 0))


def _full_spec(r, cols):
    return pl.BlockSpec((r, cols), lambda i: (0, 0))


_tc0 = pl.pallas_call(
    _tc0_body,
    grid=(_NGRID,),
    in_specs=[_row_spec(_D), _full_spec(_D, _H)],
    out_specs=_row_spec(_H),
    out_shape=jax.ShapeDtypeStruct((_N, _H), jnp.float32),
)

_tc1 = pl.pallas_call(
    _tc1_body,
    grid=(_NGRID,),
    in_specs=[_row_spec(_H), _pair_spec],
    out_specs=[_row_spec(_H), _row_spec(_H)],
    out_shape=[jax.ShapeDtypeStruct((_N, _H), jnp.float32),
               jax.ShapeDtypeStruct((_N, _H), jnp.float32)],
)

_tc2 = pl.pallas_call(
    _tc2_body,
    grid=(_NGRID,),
    in_specs=[_pair_spec, _row_spec(_H), _row_spec(_H),
              _full_spec(1, _H), _full_spec(_H, _H)],
    out_specs=_row_spec(_H),
    out_shape=jax.ShapeDtypeStruct((_N, _H), jnp.float32),
)

_tc3 = pl.pallas_call(
    _tc3_body,
    grid=(_NGRID,),
    in_specs=[_pair_spec, _row_spec(_H), _row_spec(_H),
              _full_spec(1, _H), _row_spec(1), _full_spec(_H, _HID),
              _full_spec(1, _HID), _full_spec(_HID, _NC), _full_spec(1, _NC)],
    out_specs=_full_spec(_NG, _NC),
    out_shape=jax.ShapeDtypeStruct((_NG, _NC), jnp.float32),
    scratch_shapes=[pltpu.VMEM((_NG, _H), jnp.float32)],
)


def kernel(x, edge_index, batch_vec, W1, b1, W2, b2, lw1, lb1, lw2, lb2):
    src2d = edge_index[0].astype(jnp.int32).reshape(_NCHUNK, _CH)
    dst2d = edge_index[1].astype(jnp.int32).reshape(_NCHUNK, _CH)

    degp = _sc_degree(dst2d)
    h1 = _tc0(x, W1)
    hs1, dinv = _tc1(h1, degp)
    a1 = _sc_aggregate(hs1, src2d, dst2d)
    hs2 = _tc2(a1, hs1, dinv, b1.reshape(1, _H), W2)
    a2 = _sc_aggregate(hs2, src2d, dst2d)
    out = _tc3(a2, hs2, dinv, b2.reshape(1, _H),
               batch_vec.astype(jnp.int32).reshape(_N, 1), lw1,
               lb1.reshape(1, _HID), lw2, lb2.reshape(1, _NC))
    return out


# speedup vs baseline: 59.4140x; 1.4828x over previous
import functools

import jax
import jax.numpy as jnp
from jax import lax
from jax.experimental import pallas as pl
from jax.experimental.pallas import tpu as pltpu
from jax.experimental.pallas import tpu_sc as plsc

_N = 10000
_E = 320000
_D = 128
_H = 16
_HID = 100
_NG = 64
_NC = 10

_CH = 128
_NCHUNK = _E // _CH
_NW = 32
_MAXC = -(-_NCHUNK // _NW)
_BASEC = _NCHUNK // _NW
_EXTRA = _NCHUNK - _BASEC * _NW
_WIN = _MAXC
_RPS = 632
_NBUF = 8
_PREF = 6

_mesh = plsc.VectorSubcoreMesh(core_axis_name="c", subcore_axis_name="s")
_sc_params = pltpu.CompilerParams(use_tc_tiling_on_sc=False)


def _tile_ranges(w):
    cnt = jnp.where(w < _EXTRA, _MAXC, _BASEC)
    start = _BASEC * w + jnp.minimum(w, _EXTRA)
    dma_start = jnp.minimum(start, _NCHUNK - _WIN)
    loff = start - dma_start
    return cnt, dma_start, loff


def _zero_fill(ref, nrows):
    z = jnp.zeros((16,), jnp.float32)

    def body(i, _):
        ref[i] = z
        return 0

    lax.fori_loop(0, nrows, body, 0)


def _slab_start(s):
    return jnp.minimum(s * _RPS, _N - _RPS)


def _scatter_epilogue(acc_sh, out_hbm, c, s):
    plsc.subcore_barrier()
    r0 = _slab_start(s)
    pltpu.sync_copy(acc_sh.at[pl.ds(r0, _RPS)], out_hbm.at[c, pl.ds(r0, _RPS)])


@functools.partial(
    pl.kernel,
    out_type=jax.ShapeDtypeStruct((2, _N, _H), jnp.float32),
    mesh=_mesh,
    scratch_types=[
        pltpu.VMEM((_WIN, _CH), jnp.int32),
        pltpu.VMEM((_CH, _H), jnp.float32),
        pltpu.VMEM((_RPS, _H), jnp.float32),
        pltpu.VMEM_SHARED((_N, _H), jnp.float32),
        pltpu.SemaphoreType.DMA,
    ],
    compiler_params=_sc_params,
)
def _sc_degree(ei_hbm, out_hbm, dst_v, ones_v, zrows_v, acc_sh, sem):
    c = lax.axis_index("c")
    s = lax.axis_index("s")
    w = s * 2 + c
    cnt, dma_start, loff = _tile_ranges(w)

    _zero_fill(zrows_v, _RPS)
    one = jnp.ones((16,), jnp.float32)

    def fill_ones(i, _):
        ones_v[i] = one
        return 0

    lax.fori_loop(0, _CH, fill_ones, 0)
    pltpu.sync_copy(zrows_v, acc_sh.at[pl.ds(_slab_start(s), _RPS)])
    pltpu.async_copy(ei_hbm.at[1, pl.ds(dma_start, _WIN)], dst_v, sem).wait()
    plsc.subcore_barrier()

    def step(k, _):
        pltpu.async_copy(ones_v, acc_sh.at[dst_v.at[loff + k]], sem, add=True)
        return 0

    lax.fori_loop(0, cnt, step, 0)

    def drain(k, _):
        pltpu.make_async_copy(ones_v, acc_sh.at[dst_v.at[loff + k]], sem).wait()
        return 0

    lax.fori_loop(0, cnt, drain, 0)
    _scatter_epilogue(acc_sh, out_hbm, c, s)


@functools.partial(
    pl.kernel,
    out_type=jax.ShapeDtypeStruct((2, _N, _H), jnp.float32),
    mesh=_mesh,
    scratch_types=[
        pltpu.VMEM((_WIN, _CH), jnp.int32),
        pltpu.VMEM((_WIN, _CH), jnp.int32),
        pltpu.VMEM((_NBUF, _CH, _H), jnp.float32),
        pltpu.VMEM((_RPS, _H), jnp.float32),
        pltpu.VMEM_SHARED((_N, _H), jnp.float32),
        pltpu.SemaphoreType.DMA,
        pltpu.SemaphoreType.DMA,
        pltpu.SemaphoreType.DMA,
    ],
    compiler_params=_sc_params,
)
def _sc_aggregate(hs_hbm, ei_hbm, out_hbm, src_v, dst_v, rows_v,
                  zrows_v, acc_sh, gsem, ssem, isem):
    c = lax.axis_index("c")
    s = lax.axis_index("s")
    w = s * 2 + c
    cnt, dma_start, loff = _tile_ranges(w)

    _zero_fill(zrows_v, _RPS)
    pltpu.sync_copy(zrows_v, acc_sh.at[pl.ds(_slab_start(s), _RPS)])
    src_win = ei_hbm.at[0, pl.ds(dma_start, _WIN)]
    dst_win = ei_hbm.at[1, pl.ds(dma_start, _WIN)]
    pltpu.async_copy(src_win, src_v, isem)
    pltpu.async_copy(dst_win, dst_v, isem)
    pltpu.make_async_copy(src_win, src_v, isem).wait()
    pltpu.make_async_copy(dst_win, dst_v, isem).wait()
    plsc.subcore_barrier()

    def gissue(j):
        pltpu.async_copy(hs_hbm.at[src_v.at[loff + j]],
                         rows_v.at[j % _NBUF], gsem)

    def gwait(j):
        pltpu.make_async_copy(hs_hbm.at[src_v.at[loff + j]],
                              rows_v.at[j % _NBUF], gsem).wait()

    def sissue(j):
        pltpu.async_copy(rows_v.at[j % _NBUF],
                         acc_sh.at[dst_v.at[loff + j]], ssem, add=True)

    def swait(j):
        pltpu.make_async_copy(rows_v.at[j % _NBUF],
                              acc_sh.at[dst_v.at[loff + j]], ssem).wait()

    for j in range(_PREF):
        gissue(j)

    def step(k, _):
        gwait(k)
        sissue(k)

        @pl.when(jnp.logical_and(k < _NBUF - _PREF, k + _PREF < cnt))
        def _():
            gissue(k + _PREF)

        @pl.when(jnp.logical_and(k >= _NBUF - _PREF, k + _PREF < cnt))
        def _():
            swait(k - (_NBUF - _PREF))
            gissue(k + _PREF)

        return 0

    lax.fori_loop(0, cnt, step, 0)

    def drain(j, _):
        swait(j)
        return 0

    lax.fori_loop(cnt - _NBUF, cnt, drain, 0)
    _scatter_epilogue(acc_sh, out_hbm, c, s)


_RB = 1000
_NGRID = _N // _RB

_hp = jax.lax.Precision.HIGHEST


def _tc0_body(x_ref, w1_ref, h_ref):
    h_ref[...] = jnp.dot(x_ref[...], w1_ref[...],
                         preferred_element_type=jnp.float32, precision=_hp)


def _tc1_body(h_ref, dp_ref, hs_ref, dinv_ref):
    deg = 1.0 + dp_ref[0] + dp_ref[1]
    dinv = lax.rsqrt(deg)
    hs_ref[...] = h_ref[...] * dinv
    dinv_ref[...] = dinv


def _tc2_body(ap_ref, hs_ref, dinv_ref, b1_ref, w2_ref, hs2_ref):
    dinv = dinv_ref[...]
    t = dinv * (ap_ref[0] + ap_ref[1] + hs_ref[...]) + b1_ref[...]
    t = jnp.maximum(t, 0.0)
    hs2_ref[...] = jnp.dot(t, w2_ref[...], preferred_element_type=jnp.float32,
                           precision=_hp) * dinv


def _tc3_body(ap_ref, hs_ref, dinv_ref, b2_ref, bv_ref, lw1_ref,
              lb1_ref, lw2_ref, lb2_ref, out_ref, acc_ref):
    i = pl.program_id(0)
    t = dinv_ref[...] * (ap_ref[0] + ap_ref[1] + hs_ref[...]) + b2_ref[...]
    t = jnp.maximum(t, 0.0)
    gids = lax.broadcasted_iota(jnp.int32, (_NG, _RB), 0)
    oh = (bv_ref[0] == gids).astype(jnp.float32)
    contrib = jnp.dot(oh, t, preferred_element_type=jnp.float32,
                      precision=_hp)

    @pl.when(i == 0)
    def _():
        acc_ref[...] = contrib

    @pl.when(i > 0)
    def _():
        acc_ref[...] = acc_ref[...] + contrib

    @pl.when(i == _NGRID - 1)
    def _():
        r = jnp.maximum(acc_ref[...], 0.0)
        q = jnp.maximum(
            jnp.dot(r, lw1_ref[...], preferred_element_type=jnp.float32,
                    precision=_hp) + lb1_ref[...], 0.0)
        out_ref[...] = jnp.dot(q, lw2_ref[...],
                               preferred_element_type=jnp.float32,
                               precision=_hp) + lb2_ref[...]


def _row_spec(cols):
    return pl.BlockSpec((_RB, cols), lambda i: (i, 0))


_pair_spec = pl.BlockSpec((2, _RB, _H), lambda i: (0, i, 0))


def _full_spec(r, cols):
    return pl.BlockSpec((r, cols), lambda i: (0, 0))


_tc0 = pl.pallas_call(
    _tc0_body,
    grid=(_NGRID,),
    in_specs=[_row_spec(_D), _full_spec(_D, _H)],
    out_specs=_row_spec(_H),
    out_shape=jax.ShapeDtypeStruct((_N, _H), jnp.float32),
)

_tc1 = pl.pallas_call(
    _tc1_body,
    grid=(_NGRID,),
    in_specs=[_row_spec(_H), _pair_spec],
    out_specs=[_row_spec(_H), _row_spec(_H)],
    out_shape=[jax.ShapeDtypeStruct((_N, _H), jnp.float32),
               jax.ShapeDtypeStruct((_N, _H), jnp.float32)],
)

_tc2 = pl.pallas_call(
    _tc2_body,
    grid=(_NGRID,),
    in_specs=[_pair_spec, _row_spec(_H), _row_spec(_H),
              _full_spec(1, _H), _full_spec(_H, _H)],
    out_specs=_row_spec(_H),
    out_shape=jax.ShapeDtypeStruct((_N, _H), jnp.float32),
)

_tc3 = pl.pallas_call(
    _tc3_body,
    grid=(_NGRID,),
    in_specs=[_pair_spec, _row_spec(_H), _row_spec(_H),
              _full_spec(1, _H), pl.BlockSpec((1, 1, _RB), lambda i: (i, 0, 0)),
              _full_spec(_H, _HID),
              _full_spec(1, _HID), _full_spec(_HID, _NC), _full_spec(1, _NC)],
    out_specs=_full_spec(_NG, _NC),
    out_shape=jax.ShapeDtypeStruct((_NG, _NC), jnp.float32),
    scratch_shapes=[pltpu.VMEM((_NG, _H), jnp.float32)],
)


def kernel(x, edge_index, batch_vec, W1, b1, W2, b2, lw1, lb1, lw2, lb2):
    ei = edge_index.astype(jnp.int32).reshape(2, _NCHUNK, _CH)

    degp = _sc_degree(ei)
    h1 = _tc0(x, W1)
    hs1, dinv = _tc1(h1, degp)
    a1 = _sc_aggregate(hs1, ei)
    hs2 = _tc2(a1, hs1, dinv, b1.reshape(1, _H), W2)
    a2 = _sc_aggregate(hs2, ei)
    out = _tc3(a2, hs2, dinv, b2.reshape(1, _H),
               batch_vec.astype(jnp.int32).reshape(_NGRID, 1, _RB), lw1,
               lb1.reshape(1, _HID), lw2, lb2.reshape(1, _NC))
    return out


# speedup vs baseline: 91.2574x; 1.5360x over previous
import functools

import jax
import jax.numpy as jnp
from jax import lax
from jax.experimental import pallas as pl
from jax.experimental.pallas import tpu as pltpu
from jax.experimental.pallas import tpu_sc as plsc

_N = 10000
_E = 320000
_D = 128
_H = 16
_HID = 100
_NG = 64
_NC = 10

_CH = 128
_NCHUNK = _E // _CH
_NW = 32
_MAXC = -(-_NCHUNK // _NW)
_BASEC = _NCHUNK // _NW
_EXTRA = _NCHUNK - _BASEC * _NW
_WIN = _MAXC
_NP = 10048
_PR = _NP // 8
_RPS = _NP // 16
_NBUF = 8
_PREF = 6

_mesh = plsc.VectorSubcoreMesh(core_axis_name="c", subcore_axis_name="s")
_sc_params = pltpu.CompilerParams(use_tc_tiling_on_sc=False)


def _tile_ranges(w):
    cnt = jnp.where(w < _EXTRA, _MAXC, _BASEC)
    start = _BASEC * w + jnp.minimum(w, _EXTRA)
    dma_start = jnp.minimum(start, _NCHUNK - _WIN)
    loff = start - dma_start
    return cnt, dma_start, loff


def _zero_fill(ref, nrows):
    z = jnp.zeros((16,), jnp.float32)

    def body(i, _):
        ref[i] = z
        return 0

    lax.fori_loop(0, nrows, body, 0)


def _scatter_epilogue(acc_sh, out_hbm, c, s):
    plsc.subcore_barrier()
    r0 = s * _RPS
    pltpu.sync_copy(acc_sh.at[pl.ds(r0, _RPS)], out_hbm.at[c, pl.ds(r0, _RPS)])


@functools.partial(
    pl.kernel,
    out_type=jax.ShapeDtypeStruct((2, _NP, _H), jnp.float32),
    mesh=_mesh,
    scratch_types=[
        pltpu.VMEM((_WIN, _CH), jnp.int32),
        pltpu.VMEM((_CH, _H), jnp.float32),
        pltpu.VMEM((_RPS, _H), jnp.float32),
        pltpu.VMEM_SHARED((_NP, _H), jnp.float32),
        pltpu.SemaphoreType.DMA,
    ],
    compiler_params=_sc_params,
)
def _sc_degree(ei_hbm, out_hbm, dst_v, ones_v, zrows_v, acc_sh, sem):
    c = lax.axis_index("c")
    s = lax.axis_index("s")
    w = s * 2 + c
    cnt, dma_start, loff = _tile_ranges(w)

    _zero_fill(zrows_v, _RPS)
    one = jnp.ones((16,), jnp.float32)

    def fill_ones(i, _):
        ones_v[i] = one
        return 0

    lax.fori_loop(0, _CH, fill_ones, 0)
    pltpu.sync_copy(zrows_v, acc_sh.at[pl.ds(s * _RPS, _RPS)])
    pltpu.async_copy(ei_hbm.at[1, pl.ds(dma_start, _WIN)], dst_v, sem).wait()
    plsc.subcore_barrier()

    def step(k, _):
        pltpu.async_copy(ones_v, acc_sh.at[dst_v.at[loff + k]], sem, add=True)
        return 0

    lax.fori_loop(0, cnt, step, 0)

    def drain(k, _):
        pltpu.make_async_copy(ones_v, acc_sh.at[dst_v.at[loff + k]], sem).wait()
        return 0

    lax.fori_loop(0, cnt, drain, 0)
    _scatter_epilogue(acc_sh, out_hbm, c, s)


@functools.partial(
    pl.kernel,
    out_type=jax.ShapeDtypeStruct((2, _NP, _H), jnp.float32),
    mesh=_mesh,
    scratch_types=[
        pltpu.VMEM((_WIN, _CH), jnp.int32),
        pltpu.VMEM((_WIN, _CH), jnp.int32),
        pltpu.VMEM((_NBUF, _CH, _H), jnp.float32),
        pltpu.VMEM((_RPS, _H), jnp.float32),
        pltpu.VMEM_SHARED((_NP, _H), jnp.float32),
        pltpu.SemaphoreType.DMA,
        pltpu.SemaphoreType.DMA,
        pltpu.SemaphoreType.DMA,
    ],
    compiler_params=_sc_params,
)
def _sc_aggregate(hs_hbm, ei_hbm, out_hbm, src_v, dst_v, rows_v,
                  zrows_v, acc_sh, gsem, ssem, isem):
    c = lax.axis_index("c")
    s = lax.axis_index("s")
    w = s * 2 + c
    cnt, dma_start, loff = _tile_ranges(w)

    _zero_fill(zrows_v, _RPS)
    pltpu.sync_copy(zrows_v, acc_sh.at[pl.ds(s * _RPS, _RPS)])
    src_win = ei_hbm.at[0, pl.ds(dma_start, _WIN)]
    dst_win = ei_hbm.at[1, pl.ds(dma_start, _WIN)]
    pltpu.async_copy(src_win, src_v, isem)
    pltpu.async_copy(dst_win, dst_v, isem)
    pltpu.make_async_copy(src_win, src_v, isem).wait()
    pltpu.make_async_copy(dst_win, dst_v, isem).wait()
    plsc.subcore_barrier()

    def gissue(j):
        pltpu.async_copy(hs_hbm.at[src_v.at[loff + j]],
                         rows_v.at[j % _NBUF], gsem)

    def gwait(j):
        pltpu.make_async_copy(hs_hbm.at[src_v.at[loff + j]],
                              rows_v.at[j % _NBUF], gsem).wait()

    def sissue(j):
        pltpu.async_copy(rows_v.at[j % _NBUF],
                         acc_sh.at[dst_v.at[loff + j]], ssem, add=True)

    def swait(j):
        pltpu.make_async_copy(rows_v.at[j % _NBUF],
                              acc_sh.at[dst_v.at[loff + j]], ssem).wait()

    for j in range(_PREF):
        gissue(j)

    def step(k, _):
        gwait(k)
        sissue(k)

        @pl.when(jnp.logical_and(k < _NBUF - _PREF, k + _PREF < cnt))
        def _():
            gissue(k + _PREF)

        @pl.when(jnp.logical_and(k >= _NBUF - _PREF, k + _PREF < cnt))
        def _():
            swait(k - (_NBUF - _PREF))
            gissue(k + _PREF)

        return 0

    lax.fori_loop(0, cnt, step, 0)

    def drain(j, _):
        swait(j)
        return 0

    lax.fori_loop(cnt - _NBUF, cnt, drain, 0)
    _scatter_epilogue(acc_sh, out_hbm, c, s)


_RB = _PR
_NB = _RB * 8
_NGRID = _PR // _RB

_hp = jax.lax.Precision.HIGHEST


def _tc0_body(xb_ref, w1bd_ref, h_ref):
    h_ref[...] = jnp.dot(xb_ref[...], w1bd_ref[...],
                         preferred_element_type=jnp.float32, precision=_hp)


def _tc1_body(h_ref, dp_ref, hs_ref, dinv_ref):
    deg = 1.0 + dp_ref[0] + dp_ref[1]
    dinv = lax.rsqrt(deg)
    hs_ref[...] = h_ref[...] * dinv
    dinv_ref[...] = dinv


def _tc2_body(ap_ref, hs_ref, dinv_ref, b1_ref, w2bd_ref, hs2_ref):
    dinv = dinv_ref[...]
    t = dinv * (ap_ref[0] + ap_ref[1] + hs_ref[...]) + b1_ref[...]
    t = jnp.maximum(t, 0.0)
    hs2_ref[...] = jnp.dot(t, w2bd_ref[...], preferred_element_type=jnp.float32,
                           precision=_hp) * dinv


def _tc3_body(ap_ref, hs_ref, dinv_ref, b2_ref, bvp_ref, fold_ref, lw1_ref,
              lb1_ref, lw2_ref, lb2_ref, out_ref):
    t = dinv_ref[...] * (ap_ref[0] + ap_ref[1] + hs_ref[...]) + b2_ref[...]
    t = jnp.maximum(t, 0.0)
    lane = lax.broadcasted_iota(jnp.int32, (_PR, 128), 1)
    prow = lax.broadcasted_iota(jnp.int32, (_PR, 128), 0)
    nid = prow * 8 + lane // 16
    t = jnp.where(nid < _N, t, 0.0)
    gids = lax.broadcasted_iota(jnp.int32, (_NG, _PR), 0)
    glane = lax.broadcasted_iota(jnp.int32, (_NG, 128), 1) // _H
    contrib = jnp.zeros((_NG, 128), jnp.float32)
    for a in range(8):
        oh = (bvp_ref[a].reshape(1, _PR) == gids).astype(jnp.float32)
        part = jnp.dot(oh, t, preferred_element_type=jnp.float32,
                       precision=_hp)
        contrib = contrib + jnp.where(glane == a, part, 0.0)
    pooled = jnp.dot(contrib, fold_ref[...], preferred_element_type=jnp.float32,
                     precision=_hp)
    r = jnp.maximum(pooled, 0.0)
    q = jnp.maximum(
        jnp.dot(r, lw1_ref[...], preferred_element_type=jnp.float32,
                precision=_hp) + lb1_ref[...], 0.0)
    out_ref[...] = jnp.dot(q, lw2_ref[...],
                           preferred_element_type=jnp.float32,
                           precision=_hp) + lb2_ref[...]


def _prow_spec():
    return pl.BlockSpec((_RB, 128), lambda i: (i, 0))


_pair_spec = pl.BlockSpec((2, _RB, 128), lambda i: (0, i, 0))


def _full_spec(r, cols):
    return pl.BlockSpec((r, cols), lambda i: (0, 0))


_pshape = jax.ShapeDtypeStruct((_PR, 128), jnp.float32)

_tc0 = pl.pallas_call(
    _tc0_body,
    grid=(_NGRID,),
    in_specs=[pl.BlockSpec((_RB, 8 * _D), lambda i: (i, 0)),
              _full_spec(8 * _D, 128)],
    out_specs=_prow_spec(),
    out_shape=_pshape,
)

_tc1 = pl.pallas_call(
    _tc1_body,
    grid=(_NGRID,),
    in_specs=[_prow_spec(), _pair_spec],
    out_specs=[_prow_spec(), _prow_spec()],
    out_shape=[_pshape, _pshape],
)

_tc2 = pl.pallas_call(
    _tc2_body,
    grid=(_NGRID,),
    in_specs=[_pair_spec, _prow_spec(), _prow_spec(),
              _full_spec(1, 128), _full_spec(128, 128)],
    out_specs=_prow_spec(),
    out_shape=_pshape,
)

_tc3 = pl.pallas_call(
    _tc3_body,
    grid=(_NGRID,),
    in_specs=[_pair_spec, _prow_spec(), _prow_spec(),
              _full_spec(1, 128), _full_spec(8, _PR), _full_spec(128, _H),
              _full_spec(_H, _HID),
              _full_spec(1, _HID), _full_spec(_HID, _NC), _full_spec(1, _NC)],
    out_specs=_full_spec(_NG, _NC),
    out_shape=jax.ShapeDtypeStruct((_NG, _NC), jnp.float32),
)


def kernel(x, edge_index, batch_vec, W1, b1, W2, b2, lw1, lb1, lw2, lb2):
    ei = edge_index.astype(jnp.int32).reshape(2, _NCHUNK, _CH)
    eye8 = jnp.eye(8, dtype=jnp.float32)
    b1p = jnp.tile(b1.reshape(1, _H), (1, 8))
    b2p = jnp.tile(b2.reshape(1, _H), (1, 8))
    w1bd = jnp.kron(eye8, W1)
    w2bd = jnp.kron(eye8, W2)
    fold = jnp.tile(jnp.eye(_H, dtype=jnp.float32), (8, 1))
    xb = jnp.pad(x, ((0, _NP - _N), (0, 0))).reshape(_PR, 8 * _D)
    bvp = jnp.pad(batch_vec.astype(jnp.int32), (0, _NP - _N),
                  constant_values=-1).reshape(_PR, 8).T

    degp = _sc_degree(ei)
    h1 = _tc0(xb, w1bd)
    hs1, dinv = _tc1(h1, degp.reshape(2, _PR, 128))
    a1 = _sc_aggregate(hs1.reshape(_NP, _H), ei)
    hs2 = _tc2(a1.reshape(2, _PR, 128), hs1, dinv, b1p, w2bd)
    a2 = _sc_aggregate(hs2.reshape(_NP, _H), ei)
    out = _tc3(a2.reshape(2, _PR, 128), hs2, dinv, b2p, bvp, fold, lw1,
               lb1.reshape(1, _HID), lw2, lb2.reshape(1, _NC))
    return out


# speedup vs baseline: 102.1094x; 1.1189x over previous
import functools

import jax
import jax.numpy as jnp
from jax import lax
from jax.experimental import pallas as pl
from jax.experimental.pallas import tpu as pltpu
from jax.experimental.pallas import tpu_sc as plsc

_N = 10000
_E = 320000
_D = 128
_H = 16
_HID = 100
_NG = 64
_NC = 10

_CH = 128
_NCHUNK = _E // _CH
_NW = 32
_MAXC = -(-_NCHUNK // _NW)
_BASEC = _NCHUNK // _NW
_EXTRA = _NCHUNK - _BASEC * _NW
_WIN = _MAXC
_NP = 10048
_PR = _NP // 8
_RPS = _NP // 16
_NBUF = 12
_PREF = 10

_mesh = plsc.VectorSubcoreMesh(core_axis_name="c", subcore_axis_name="s")
_sc_params = pltpu.CompilerParams(use_tc_tiling_on_sc=False)


def _tile_ranges(w):
    cnt = jnp.where(w < _EXTRA, _MAXC, _BASEC)
    start = _BASEC * w + jnp.minimum(w, _EXTRA)
    dma_start = jnp.minimum(start, _NCHUNK - _WIN)
    loff = start - dma_start
    return cnt, dma_start, loff


def _zero_fill(ref, nrows):
    z = jnp.zeros((16,), jnp.float32)

    def body(i, _):
        ref[i] = z
        return 0

    lax.fori_loop(0, nrows, body, 0)


def _scatter_epilogue(acc_sh, out_hbm, c, s):
    plsc.subcore_barrier()
    r0 = s * _RPS
    pltpu.sync_copy(acc_sh.at[pl.ds(r0, _RPS)], out_hbm.at[c, pl.ds(r0, _RPS)])


@functools.partial(
    pl.kernel,
    out_type=jax.ShapeDtypeStruct((2, _NP, _H), jnp.float32),
    mesh=_mesh,
    scratch_types=[
        pltpu.VMEM((_WIN * _CH,), jnp.int32),
        pltpu.VMEM((_CH, _H), jnp.float32),
        pltpu.VMEM((_RPS, _H), jnp.float32),
        pltpu.VMEM_SHARED((_NP, _H), jnp.float32),
        pltpu.SemaphoreType.DMA,
    ],
    compiler_params=_sc_params,
)
def _sc_degree(ei_hbm, out_hbm, dst_v, ones_v, zrows_v, acc_sh, sem):
    c = lax.axis_index("c")
    s = lax.axis_index("s")
    w = s * 2 + c
    cnt, dma_start, loff = _tile_ranges(w)

    _zero_fill(zrows_v, _RPS)
    one = jnp.ones((16,), jnp.float32)

    def fill_ones(i, _):
        ones_v[i] = one
        return 0

    lax.fori_loop(0, _CH, fill_ones, 0)
    pltpu.sync_copy(zrows_v, acc_sh.at[pl.ds(s * _RPS, _RPS)])
    pltpu.async_copy(ei_hbm.at[1, pl.ds(dma_start * _CH, _WIN * _CH)], dst_v,
                     sem).wait()
    plsc.subcore_barrier()

    def step(k, _):
        idx = dst_v.at[pl.ds((loff + k) * _CH, _CH)]
        pltpu.async_copy(ones_v, acc_sh.at[idx], sem, add=True)
        return 0

    lax.fori_loop(0, cnt, step, 0)

    def drain(k, _):
        idx = dst_v.at[pl.ds((loff + k) * _CH, _CH)]
        pltpu.make_async_copy(ones_v, acc_sh.at[idx], sem).wait()
        return 0

    lax.fori_loop(0, cnt, drain, 0)
    _scatter_epilogue(acc_sh, out_hbm, c, s)


@functools.partial(
    pl.kernel,
    out_type=jax.ShapeDtypeStruct((2, _NP, _H), jnp.float32),
    mesh=_mesh,
    scratch_types=[
        pltpu.VMEM((_WIN * _CH,), jnp.int32),
        pltpu.VMEM((_WIN * _CH,), jnp.int32),
        pltpu.VMEM((_NBUF, _CH, _H), jnp.float32),
        pltpu.VMEM((_RPS, _H), jnp.float32),
        pltpu.VMEM_SHARED((_NP, _H), jnp.float32),
        pltpu.SemaphoreType.DMA,
        pltpu.SemaphoreType.DMA,
        pltpu.SemaphoreType.DMA,
    ],
    compiler_params=_sc_params,
)
def _sc_aggregate(hs_hbm, ei_hbm, out_hbm, src_v, dst_v, rows_v,
                  zrows_v, acc_sh, gsem, ssem, isem):
    c = lax.axis_index("c")
    s = lax.axis_index("s")
    w = s * 2 + c
    cnt, dma_start, loff = _tile_ranges(w)

    _zero_fill(zrows_v, _RPS)
    pltpu.sync_copy(zrows_v, acc_sh.at[pl.ds(s * _RPS, _RPS)])
    src_win = ei_hbm.at[0, pl.ds(dma_start * _CH, _WIN * _CH)]
    dst_win = ei_hbm.at[1, pl.ds(dma_start * _CH, _WIN * _CH)]
    pltpu.async_copy(src_win, src_v, isem)
    pltpu.async_copy(dst_win, dst_v, isem)
    pltpu.make_async_copy(src_win, src_v, isem).wait()
    pltpu.make_async_copy(dst_win, dst_v, isem).wait()
    plsc.subcore_barrier()

    def _sidx(j):
        return src_v.at[pl.ds((loff + j) * _CH, _CH)]

    def _didx(j):
        return dst_v.at[pl.ds((loff + j) * _CH, _CH)]

    def gissue(j):
        pltpu.async_copy(hs_hbm.at[_sidx(j)], rows_v.at[j % _NBUF], gsem)

    def gwait(j):
        pltpu.make_async_copy(hs_hbm.at[_sidx(j)],
                              rows_v.at[j % _NBUF], gsem).wait()

    def sissue(j):
        pltpu.async_copy(rows_v.at[j % _NBUF], acc_sh.at[_didx(j)], ssem,
                         add=True)

    def swait(j):
        pltpu.make_async_copy(rows_v.at[j % _NBUF],
                              acc_sh.at[_didx(j)], ssem).wait()

    for j in range(_PREF):
        gissue(j)

    def step(k, _):
        gwait(k)
        sissue(k)

        @pl.when(jnp.logical_and(k < _NBUF - _PREF, k + _PREF < cnt))
        def _():
            gissue(k + _PREF)

        @pl.when(jnp.logical_and(k >= _NBUF - _PREF, k + _PREF < cnt))
        def _():
            swait(k - (_NBUF - _PREF))
            gissue(k + _PREF)

        return 0

    lax.fori_loop(0, cnt, step, 0)

    def drain(j, _):
        swait(j)
        return 0

    lax.fori_loop(cnt - _NBUF, cnt, drain, 0)
    _scatter_epilogue(acc_sh, out_hbm, c, s)


_RB = _PR
_NB = _RB * 8
_NGRID = _PR // _RB

_hp = jax.lax.Precision.HIGHEST


def _tc0_body(xb_ref, w1bd_ref, h_ref):
    h_ref[...] = jnp.dot(xb_ref[...], w1bd_ref[...],
                         preferred_element_type=jnp.float32, precision=_hp)


def _tc1_body(h_ref, dp_ref, hs_ref, dinv_ref):
    deg = 1.0 + dp_ref[0] + dp_ref[1]
    dinv = lax.rsqrt(deg)
    hs_ref[...] = h_ref[...] * dinv
    dinv_ref[...] = dinv


def _tc2_body(ap_ref, hs_ref, dinv_ref, b1_ref, w2bd_ref, hs2_ref):
    dinv = dinv_ref[...]
    t = dinv * (ap_ref[0] + ap_ref[1] + hs_ref[...]) + b1_ref[...]
    t = jnp.maximum(t, 0.0)
    hs2_ref[...] = jnp.dot(t, w2bd_ref[...], preferred_element_type=jnp.float32,
                           precision=_hp) * dinv


def _tc3_body(ap_ref, hs_ref, dinv_ref, b2_ref, bvp_ref, fold_ref, lw1_ref,
              lb1_ref, lw2_ref, lb2_ref, out_ref):
    t = dinv_ref[...] * (ap_ref[0] + ap_ref[1] + hs_ref[...]) + b2_ref[...]
    t = jnp.maximum(t, 0.0)
    lane = lax.broadcasted_iota(jnp.int32, (_PR, 128), 1)
    prow = lax.broadcasted_iota(jnp.int32, (_PR, 128), 0)
    nid = prow * 8 + lane // 16
    t = jnp.where(nid < _N, t, 0.0)
    gids = lax.broadcasted_iota(jnp.int32, (_NG, _PR), 0)
    glane = lax.broadcasted_iota(jnp.int32, (_NG, 128), 1) // _H
    contrib = jnp.zeros((_NG, 128), jnp.float32)
    for a in range(8):
        oh = (bvp_ref[a].reshape(1, _PR) == gids).astype(jnp.float32)
        part = jnp.dot(oh, t, preferred_element_type=jnp.float32,
                       precision=_hp)
        contrib = contrib + jnp.where(glane == a, part, 0.0)
    pooled = jnp.dot(contrib, fold_ref[...], preferred_element_type=jnp.float32,
                     precision=_hp)
    r = jnp.maximum(pooled, 0.0)
    q = jnp.maximum(
        jnp.dot(r, lw1_ref[...], preferred_element_type=jnp.float32,
                precision=_hp) + lb1_ref[...], 0.0)
    out_ref[...] = jnp.dot(q, lw2_ref[...],
                           preferred_element_type=jnp.float32,
                           precision=_hp) + lb2_ref[...]


def _prow_spec():
    return pl.BlockSpec((_RB, 128), lambda i: (i, 0))


_pair_spec = pl.BlockSpec((2, _RB, 128), lambda i: (0, i, 0))


def _full_spec(r, cols):
    return pl.BlockSpec((r, cols), lambda i: (0, 0))


_pshape = jax.ShapeDtypeStruct((_PR, 128), jnp.float32)

_tc0 = pl.pallas_call(
    _tc0_body,
    grid=(_NGRID,),
    in_specs=[pl.BlockSpec((_RB, 8 * _D), lambda i: (i, 0)),
              _full_spec(8 * _D, 128)],
    out_specs=_prow_spec(),
    out_shape=_pshape,
)

_tc1 = pl.pallas_call(
    _tc1_body,
    grid=(_NGRID,),
    in_specs=[_prow_spec(), _pair_spec],
    out_specs=[_prow_spec(), _prow_spec()],
    out_shape=[_pshape, _pshape],
)

_tc2 = pl.pallas_call(
    _tc2_body,
    grid=(_NGRID,),
    in_specs=[_pair_spec, _prow_spec(), _prow_spec(),
              _full_spec(1, 128), _full_spec(128, 128)],
    out_specs=_prow_spec(),
    out_shape=_pshape,
)

_tc3 = pl.pallas_call(
    _tc3_body,
    grid=(_NGRID,),
    in_specs=[_pair_spec, _prow_spec(), _prow_spec(),
              _full_spec(1, 128), _full_spec(8, _PR), _full_spec(128, _H),
              _full_spec(_H, _HID),
              _full_spec(1, _HID), _full_spec(_HID, _NC), _full_spec(1, _NC)],
    out_specs=_full_spec(_NG, _NC),
    out_shape=jax.ShapeDtypeStruct((_NG, _NC), jnp.float32),
)


def kernel(x, edge_index, batch_vec, W1, b1, W2, b2, lw1, lb1, lw2, lb2):
    ei = edge_index.astype(jnp.int32)
    eye8 = jnp.eye(8, dtype=jnp.float32)
    b1p = jnp.tile(b1.reshape(1, _H), (1, 8))
    b2p = jnp.tile(b2.reshape(1, _H), (1, 8))
    w1bd = jnp.kron(eye8, W1)
    w2bd = jnp.kron(eye8, W2)
    fold = jnp.tile(jnp.eye(_H, dtype=jnp.float32), (8, 1))
    xb = jnp.pad(x, ((0, _NP - _N), (0, 0))).reshape(_PR, 8 * _D)
    bvp = jnp.pad(batch_vec.astype(jnp.int32), (0, _NP - _N),
                  constant_values=-1).reshape(_PR, 8).T

    degp = _sc_degree(ei)
    h1 = _tc0(xb, w1bd)
    hs1, dinv = _tc1(h1, degp.reshape(2, _PR, 128))
    a1 = _sc_aggregate(hs1.reshape(_NP, _H), ei)
    hs2 = _tc2(a1.reshape(2, _PR, 128), hs1, dinv, b1p, w2bd)
    a2 = _sc_aggregate(hs2.reshape(_NP, _H), ei)
    out = _tc3(a2.reshape(2, _PR, 128), hs2, dinv, b2p, bvp, fold, lw1,
               lb1.reshape(1, _HID), lw2, lb2.reshape(1, _NC))
    return out


# speedup vs baseline: 102.7476x; 1.0063x over previous
import functools

import jax
import jax.numpy as jnp
from jax import lax
from jax.experimental import pallas as pl
from jax.experimental.pallas import tpu as pltpu
from jax.experimental.pallas import tpu_sc as plsc

_N = 10000
_E = 320000
_D = 128
_H = 16
_HID = 100
_NG = 64
_NC = 10

_CH = 256
_NCHUNK = _E // _CH
_NW = 32
_MAXC = -(-_NCHUNK // _NW)
_BASEC = _NCHUNK // _NW
_EXTRA = _NCHUNK - _BASEC * _NW
_WIN = _MAXC
_NP = 10048
_PR = _NP // 8
_RPS = _NP // 16
_NBUF = 12
_PREF = 10

_mesh = plsc.VectorSubcoreMesh(core_axis_name="c", subcore_axis_name="s")
_sc_params = pltpu.CompilerParams(use_tc_tiling_on_sc=False)


def _tile_ranges(w):
    cnt = jnp.where(w < _EXTRA, _MAXC, _BASEC)
    start = _BASEC * w + jnp.minimum(w, _EXTRA)
    dma_start = jnp.minimum(start, _NCHUNK - _WIN)
    loff = start - dma_start
    return cnt, dma_start, loff


def _zero_fill(ref, nrows):
    z = jnp.zeros((16,), jnp.float32)

    def body(i, _):
        ref[i] = z
        return 0

    lax.fori_loop(0, nrows, body, 0)


def _scatter_epilogue(acc_sh, out_hbm, c, s):
    plsc.subcore_barrier()
    r0 = s * _RPS
    pltpu.sync_copy(acc_sh.at[pl.ds(r0, _RPS)], out_hbm.at[c, pl.ds(r0, _RPS)])


@functools.partial(
    pl.kernel,
    out_type=jax.ShapeDtypeStruct((2, _NP, _H), jnp.float32),
    mesh=_mesh,
    scratch_types=[
        pltpu.VMEM((_WIN * _CH,), jnp.int32),
        pltpu.VMEM((_CH, _H), jnp.float32),
        pltpu.VMEM((_RPS, _H), jnp.float32),
        pltpu.VMEM_SHARED((_NP, _H), jnp.float32),
        pltpu.SemaphoreType.DMA,
    ],
    compiler_params=_sc_params,
)
def _sc_degree(ei_hbm, out_hbm, dst_v, ones_v, zrows_v, acc_sh, sem):
    c = lax.axis_index("c")
    s = lax.axis_index("s")
    w = s * 2 + c
    cnt, dma_start, loff = _tile_ranges(w)

    _zero_fill(zrows_v, _RPS)
    one = jnp.ones((16,), jnp.float32)

    def fill_ones(i, _):
        ones_v[i] = one
        return 0

    lax.fori_loop(0, _CH, fill_ones, 0)
    pltpu.sync_copy(zrows_v, acc_sh.at[pl.ds(s * _RPS, _RPS)])
    pltpu.async_copy(ei_hbm.at[1, pl.ds(dma_start * _CH, _WIN * _CH)], dst_v,
                     sem).wait()
    plsc.subcore_barrier()

    def step(k, _):
        idx = dst_v.at[pl.ds((loff + k) * _CH, _CH)]
        pltpu.async_copy(ones_v, acc_sh.at[idx], sem, add=True)
        return 0

    lax.fori_loop(0, cnt, step, 0)

    def drain(k, _):
        idx = dst_v.at[pl.ds((loff + k) * _CH, _CH)]
        pltpu.make_async_copy(ones_v, acc_sh.at[idx], sem).wait()
        return 0

    lax.fori_loop(0, cnt, drain, 0)
    _scatter_epilogue(acc_sh, out_hbm, c, s)


@functools.partial(
    pl.kernel,
    out_type=jax.ShapeDtypeStruct((2, _NP, _H), jnp.float32),
    mesh=_mesh,
    scratch_types=[
        pltpu.VMEM((_WIN * _CH,), jnp.int32),
        pltpu.VMEM((_WIN * _CH,), jnp.int32),
        pltpu.VMEM((_NBUF, _CH, _H), jnp.float32),
        pltpu.VMEM((_RPS, _H), jnp.float32),
        pltpu.VMEM_SHARED((_NP, _H), jnp.float32),
        pltpu.SemaphoreType.DMA,
        pltpu.SemaphoreType.DMA,
        pltpu.SemaphoreType.DMA,
    ],
    compiler_params=_sc_params,
)
def _sc_aggregate(hs_hbm, ei_hbm, out_hbm, src_v, dst_v, rows_v,
                  zrows_v, acc_sh, gsem, ssem, isem):
    c = lax.axis_index("c")
    s = lax.axis_index("s")
    w = s * 2 + c
    cnt, dma_start, loff = _tile_ranges(w)

    _zero_fill(zrows_v, _RPS)
    pltpu.sync_copy(zrows_v, acc_sh.at[pl.ds(s * _RPS, _RPS)])
    src_win = ei_hbm.at[0, pl.ds(dma_start * _CH, _WIN * _CH)]
    dst_win = ei_hbm.at[1, pl.ds(dma_start * _CH, _WIN * _CH)]
    pltpu.async_copy(src_win, src_v, isem)
    pltpu.async_copy(dst_win, dst_v, isem)
    pltpu.make_async_copy(src_win, src_v, isem).wait()
    pltpu.make_async_copy(dst_win, dst_v, isem).wait()
    plsc.subcore_barrier()

    def _sidx(j):
        return src_v.at[pl.ds((loff + j) * _CH, _CH)]

    def _didx(j):
        return dst_v.at[pl.ds((loff + j) * _CH, _CH)]

    def gissue(j):
        pltpu.async_copy(hs_hbm.at[_sidx(j)], rows_v.at[j % _NBUF], gsem)

    def gwait(j):
        pltpu.make_async_copy(hs_hbm.at[_sidx(j)],
                              rows_v.at[j % _NBUF], gsem).wait()

    def sissue(j):
        pltpu.async_copy(rows_v.at[j % _NBUF], acc_sh.at[_didx(j)], ssem,
                         add=True)

    def swait(j):
        pltpu.make_async_copy(rows_v.at[j % _NBUF],
                              acc_sh.at[_didx(j)], ssem).wait()

    for j in range(_PREF):
        gissue(j)

    def step(k, _):
        gwait(k)
        sissue(k)

        @pl.when(jnp.logical_and(k < _NBUF - _PREF, k + _PREF < cnt))
        def _():
            gissue(k + _PREF)

        @pl.when(jnp.logical_and(k >= _NBUF - _PREF, k + _PREF < cnt))
        def _():
            swait(k - (_NBUF - _PREF))
            gissue(k + _PREF)

        return 0

    lax.fori_loop(0, cnt, step, 0)

    def drain(j, _):
        swait(j)
        return 0

    lax.fori_loop(cnt - _NBUF, cnt, drain, 0)
    _scatter_epilogue(acc_sh, out_hbm, c, s)


_RB = _PR
_NB = _RB * 8
_NGRID = _PR // _RB

_hp = jax.lax.Precision.HIGHEST


def _tc0_body(xb_ref, w1bd_ref, h_ref):
    h_ref[...] = jnp.dot(xb_ref[...], w1bd_ref[...],
                         preferred_element_type=jnp.float32, precision=_hp)


def _tc1_body(h_ref, dp_ref, hs_ref, dinv_ref):
    deg = 1.0 + dp_ref[0] + dp_ref[1]
    dinv = lax.rsqrt(deg)
    hs_ref[...] = h_ref[...] * dinv
    dinv_ref[...] = dinv


def _tc2_body(ap_ref, hs_ref, dinv_ref, b1_ref, w2bd_ref, hs2_ref):
    dinv = dinv_ref[...]
    t = dinv * (ap_ref[0] + ap_ref[1] + hs_ref[...]) + b1_ref[...]
    t = jnp.maximum(t, 0.0)
    hs2_ref[...] = jnp.dot(t, w2bd_ref[...], preferred_element_type=jnp.float32,
                           precision=_hp) * dinv


def _tc3_body(ap_ref, hs_ref, dinv_ref, b2_ref, bvp_ref, fold_ref, lw1_ref,
              lb1_ref, lw2_ref, lb2_ref, out_ref):
    t = dinv_ref[...] * (ap_ref[0] + ap_ref[1] + hs_ref[...]) + b2_ref[...]
    t = jnp.maximum(t, 0.0)
    lane = lax.broadcasted_iota(jnp.int32, (_PR, 128), 1)
    prow = lax.broadcasted_iota(jnp.int32, (_PR, 128), 0)
    nid = prow * 8 + lane // 16
    t = jnp.where(nid < _N, t, 0.0)
    gids = lax.broadcasted_iota(jnp.int32, (_NG, _PR), 0)
    glane = lax.broadcasted_iota(jnp.int32, (_NG, 128), 1) // _H
    contrib = jnp.zeros((_NG, 128), jnp.float32)
    for a in range(8):
        oh = (bvp_ref[a].reshape(1, _PR) == gids).astype(jnp.float32)
        part = jnp.dot(oh, t, preferred_element_type=jnp.float32,
                       precision=_hp)
        contrib = contrib + jnp.where(glane == a, part, 0.0)
    pooled = jnp.dot(contrib, fold_ref[...], preferred_element_type=jnp.float32,
                     precision=_hp)
    r = jnp.maximum(pooled, 0.0)
    q = jnp.maximum(
        jnp.dot(r, lw1_ref[...], preferred_element_type=jnp.float32,
                precision=_hp) + lb1_ref[...], 0.0)
    out_ref[...] = jnp.dot(q, lw2_ref[...],
                           preferred_element_type=jnp.float32,
                           precision=_hp) + lb2_ref[...]


def _prow_spec():
    return pl.BlockSpec((_RB, 128), lambda i: (i, 0))


_pair_spec = pl.BlockSpec((2, _RB, 128), lambda i: (0, i, 0))


def _full_spec(r, cols):
    return pl.BlockSpec((r, cols), lambda i: (0, 0))


_pshape = jax.ShapeDtypeStruct((_PR, 128), jnp.float32)

_tc0 = pl.pallas_call(
    _tc0_body,
    grid=(_NGRID,),
    in_specs=[pl.BlockSpec((_RB, 8 * _D), lambda i: (i, 0)),
              _full_spec(8 * _D, 128)],
    out_specs=_prow_spec(),
    out_shape=_pshape,
)

_tc1 = pl.pallas_call(
    _tc1_body,
    grid=(_NGRID,),
    in_specs=[_prow_spec(), _pair_spec],
    out_specs=[_prow_spec(), _prow_spec()],
    out_shape=[_pshape, _pshape],
)

_tc2 = pl.pallas_call(
    _tc2_body,
    grid=(_NGRID,),
    in_specs=[_pair_spec, _prow_spec(), _prow_spec(),
              _full_spec(1, 128), _full_spec(128, 128)],
    out_specs=_prow_spec(),
    out_shape=_pshape,
)

_tc3 = pl.pallas_call(
    _tc3_body,
    grid=(_NGRID,),
    in_specs=[_pair_spec, _prow_spec(), _prow_spec(),
              _full_spec(1, 128), _full_spec(8, _PR), _full_spec(128, _H),
              _full_spec(_H, _HID),
              _full_spec(1, _HID), _full_spec(_HID, _NC), _full_spec(1, _NC)],
    out_specs=_full_spec(_NG, _NC),
    out_shape=jax.ShapeDtypeStruct((_NG, _NC), jnp.float32),
)


def kernel(x, edge_index, batch_vec, W1, b1, W2, b2, lw1, lb1, lw2, lb2):
    ei = edge_index.astype(jnp.int32)
    eye8 = jnp.eye(8, dtype=jnp.float32)
    b1p = jnp.tile(b1.reshape(1, _H), (1, 8))
    b2p = jnp.tile(b2.reshape(1, _H), (1, 8))
    w1bd = jnp.kron(eye8, W1)
    w2bd = jnp.kron(eye8, W2)
    fold = jnp.tile(jnp.eye(_H, dtype=jnp.float32), (8, 1))
    xb = jnp.pad(x, ((0, _NP - _N), (0, 0))).reshape(_PR, 8 * _D)
    bvp = jnp.pad(batch_vec.astype(jnp.int32), (0, _NP - _N),
                  constant_values=-1).reshape(_PR, 8).T

    degp = _sc_degree(ei)
    h1 = _tc0(xb, w1bd)
    hs1, dinv = _tc1(h1, degp.reshape(2, _PR, 128))
    a1 = _sc_aggregate(hs1.reshape(_NP, _H), ei)
    hs2 = _tc2(a1.reshape(2, _PR, 128), hs1, dinv, b1p, w2bd)
    a2 = _sc_aggregate(hs2.reshape(_NP, _H), ei)
    out = _tc3(a2.reshape(2, _PR, 128), hs2, dinv, b2p, bvp, fold, lw1,
               lb1.reshape(1, _HID), lw2, lb2.reshape(1, _NC))
    return out
